# baseline (speedup 1.0000x reference)
import functools

import jax
import jax.numpy as jnp
from jax import lax
from jax.experimental import pallas as pl
from jax.experimental.pallas import tpu as pltpu
from jax.experimental.pallas import tpu_sc as plsc

_C = 128
_SB = 8
_EPS = 1e-5


def _sc_message_pass(xk, ep, hi2, n_cols, gpt, rg):
    n, d = xk.shape
    fl_rows = (n // 16) // 8 * 8
    tail0 = fl_rows * 16
    hflat = n * n_cols

    mesh = plsc.VectorSubcoreMesh(core_axis_name="c", subcore_axis_name="s")

    @functools.partial(
        pl.kernel,
        out_type=(
            jax.ShapeDtypeStruct((2, n, d), jnp.float32),
            jax.ShapeDtypeStruct((2, hflat), jnp.float32),
        ),
        mesh=mesh,
        scratch_types=(
            pltpu.VMEM((2, _SB, _C), jnp.int32),
            pltpu.VMEM((2, _SB, _C), jnp.int32),
            pltpu.VMEM((2, _SB, _C), jnp.int32),
            pltpu.VMEM((2, _C, d), jnp.float32),
            pltpu.VMEM((_C,), jnp.float32),
            pltpu.VMEM((_C,), jnp.float32),
            pltpu.VMEM_SHARED((n, d), jnp.float32),
            pltpu.VMEM_SHARED((hflat,), jnp.float32),
            pltpu.SemaphoreType.DMA((2,)),
            pltpu.SemaphoreType.DMA((2,)),
            pltpu.SemaphoreType.DMA((2,)),
            pltpu.SemaphoreType.DMA,
        ),
    )
    def run(x_hbm, ep_hbm, hi2_hbm, zr_hbm, zh_hbm, pos_hbm, neg_hbm,
            p_hbm, h_hbm,
            src_i, dst_i, hid_i, bufs_v, pos_v, neg_v, acc_sh, hist_sh,
            gsem, ssem, isem, hsem):
        c = lax.axis_index("c")
        s = lax.axis_index("s")
        rbase = s * gpt

        pltpu.sync_copy(pos_hbm, pos_v)
        pltpu.sync_copy(neg_hbm, neg_v)

        z0 = bufs_v.at[0]
        pltpu.sync_copy(zr_hbm, z0)
        zb = s * fl_rows
        nchunks = (fl_rows + _C - 1) // _C
        for k in range(nchunks):
            m = min(_C, fl_rows - k * _C)
            pltpu.sync_copy(z0.at[pl.ds(0, m)],
                            acc_sh.at[pl.ds(zb + k * _C, m)])

        @pl.when(s == 15)
        def _zero_tail():
            left = n - tail0
            off = tail0
            while left > 0:
                mm = min(_C, left)
                pltpu.sync_copy(z0.at[pl.ds(0, mm)],
                                acc_sh.at[pl.ds(off, mm)])
                off += mm
                left -= mm

        @pl.when(s == 0)
        def _zero_hist():
            pltpu.sync_copy(zh_hbm, hist_sh)

        plsc.subcore_barrier()

        pltpu.sync_copy(ep_hbm.at[c, pl.ds(rbase, _SB)], src_i.at[0])
        pltpu.sync_copy(ep_hbm.at[1 - c, pl.ds(rbase, _SB)], dst_i.at[0])
        pltpu.sync_copy(hi2_hbm.at[c, pl.ds(rbase, _SB)], hid_i.at[0])
        pltpu.async_copy(x_hbm.at[src_i.at[0, 0]], bufs_v.at[0], gsem.at[0])

        def body(g, carry):
            sb = g // _SB
            pos = g - sb * _SB
            slot = lax.rem(sb, 2)
            b = lax.rem(g, 2)
            gg = rbase + g

            @pl.when(jnp.logical_and(pos == 0,
                                     jnp.logical_and(g + _SB < gpt,
                                                     gg + _SB < rg)))
            def _prefetch():
                nslot = 1 - slot
                nrow = rbase + (sb + 1) * _SB
                pltpu.async_copy(ep_hbm.at[c, pl.ds(nrow, _SB)],
                                 src_i.at[nslot], isem.at[nslot])
                pltpu.async_copy(ep_hbm.at[1 - c, pl.ds(nrow, _SB)],
                                 dst_i.at[nslot], isem.at[nslot])
                pltpu.async_copy(hi2_hbm.at[c, pl.ds(nrow, _SB)],
                                 hid_i.at[nslot], isem.at[nslot])

            @pl.when(gg < rg)
            def _process():
                pltpu.make_async_copy(x_hbm.at[src_i.at[slot, pos]],
                                      bufs_v.at[b], gsem.at[b]).wait()
                pltpu.async_copy(bufs_v.at[b], acc_sh.at[dst_i.at[slot, pos]],
                                 ssem.at[b], add=True)

                @pl.when(c == 0)
                def _pos_hist():
                    pltpu.async_copy(pos_v, hist_sh.at[hid_i.at[slot, pos]],
                                     hsem, add=True)

                @pl.when(c == 1)
                def _neg_hist():
                    pltpu.async_copy(neg_v, hist_sh.at[hid_i.at[slot, pos]],
                                     hsem, add=True)

            @pl.when(jnp.logical_and(g + 1 < gpt, gg + 1 < rg))
            def _next_gather():
                bn = 1 - b

                @pl.when(g > 0)
                def _wait_prev_scatter():
                    pltpu.make_async_copy(zr_hbm, bufs_v.at[bn],
                                          ssem.at[bn]).wait()

                gn = g + 1
                sbn = gn // _SB
                posn = gn - sbn * _SB
                slotn = lax.rem(sbn, 2)

                @pl.when(posn == 0)
                def _wait_idx():
                    for _ in range(3):
                        pltpu.make_async_copy(ep_hbm.at[c, pl.ds(0, _SB)],
                                              src_i.at[slotn],
                                              isem.at[slotn]).wait()

                pltpu.async_copy(x_hbm.at[src_i.at[slotn, posn]],
                                 bufs_v.at[bn], gsem.at[bn])

            return carry

        lax.fori_loop(0, gpt, body, 0)
        for b in range(2):
            pltpu.make_async_copy(zr_hbm, bufs_v.at[b], ssem.at[b]).wait()
        nreal = jnp.clip(rg - rbase, 0, gpt)

        def _drain(i, carry):
            pltpu.make_async_copy(pos_hbm, pos_v, hsem).wait()
            return carry

        lax.fori_loop(0, nreal, _drain, 0)
        plsc.subcore_barrier()

        fb = s * fl_rows
        pltpu.sync_copy(acc_sh.at[pl.ds(fb, fl_rows)],
                        p_hbm.at[c, pl.ds(fb, fl_rows)])
        pltpu.sync_copy(hist_sh.at[pl.ds(fb * n_cols, fl_rows * n_cols)],
                        h_hbm.at[c, pl.ds(fb * n_cols, fl_rows * n_cols)])

        @pl.when(s == 15)
        def _flush_tail():
            pltpu.sync_copy(acc_sh.at[pl.ds(tail0, n - tail0)],
                            p_hbm.at[c, pl.ds(tail0, n - tail0)])
            pltpu.sync_copy(
                hist_sh.at[pl.ds(tail0 * n_cols, (n - tail0) * n_cols)],
                h_hbm.at[c, pl.ds(tail0 * n_cols, (n - tail0) * n_cols)])

    zr = jnp.zeros((_C, d), jnp.float32)
    zh = jnp.zeros((hflat,), jnp.float32)
    pos1 = jnp.ones((_C,), jnp.float32)
    neg1 = jnp.full((_C,), -1.0, jnp.float32)
    return run(xk, ep, hi2, zr, zh, pos1, neg1)


def _tc_prepass(x, kernels, bn_gamma, bn_beta):
    n, d = x.shape

    def body(x_ref, k_ref, g_ref, b_ref, o_ref):
        xv = x_ref[...]
        mean = jnp.mean(xv, axis=0)
        xc = xv - mean[None, :]
        var = jnp.mean(xc * xc, axis=0)
        sc = g_ref[0, :] / jnp.sqrt(var + _EPS)
        o_ref[...] = (xc * sc[None, :] + b_ref[0, :][None, :]) * k_ref[0, :]

    return pl.pallas_call(
        body,
        out_shape=jax.ShapeDtypeStruct((n, d), jnp.float32),
    )(x, kernels, bn_gamma, bn_beta)


def _tc_combine(xk, p, h, r, kernels, escal):
    n, d = xk.shape

    def body(x_ref, p_ref, h_ref, r_ref, k_ref, es_ref, o_ref):
        rk = r_ref[...] * k_ref[0, :][None, :]
        pv = p_ref[0] + p_ref[1]
        hv = h_ref[0] + h_ref[1]
        relpart = jnp.dot(hv, rk, preferred_element_type=jnp.float32,
                          precision=lax.Precision.HIGHEST)
        num = pv + relpart + x_ref[...]
        esv = es_ref[...]
        iot = lax.broadcasted_iota(jnp.int32, (n, 1), 0)
        du = jnp.ones((n, 1), jnp.float32)
        for i in range(3):
            ai = esv[2 * i:2 * i + 1, :]
            bi = esv[2 * i + 1:2 * i + 2, :]
            inc = (ai != bi).astype(jnp.float32)
            du = du + inc * ((iot == ai).astype(jnp.float32)
                             + (iot == bi).astype(jnp.float32))
        o_ref[...] = num / du

    return pl.pallas_call(
        body,
        out_shape=jax.ShapeDtypeStruct((n, d), jnp.float32),
    )(xk, p, h, r, kernels, escal)


def kernel(x, edges, rels, r, kernels, bn_gamma, bn_beta):
    n, d = x.shape
    e = edges.shape[1]
    nrel = r.shape[0]
    rg = e // _C
    rpad = -(-rg // _SB) * _SB - rg
    gpt = -(-rg // (16 * _SB)) * _SB
    e2 = edges.astype(jnp.int32).reshape(2, rg, _C)
    rl2 = rels.astype(jnp.int32).reshape(1, rg, _C)
    hi2 = jnp.concatenate([e2[1:2], e2[0:1]]) * nrel + rl2
    zpad = jnp.zeros((2, rpad, _C), jnp.int32)
    ep = jnp.concatenate([e2, zpad], axis=1)
    hi2 = jnp.concatenate([hi2, zpad], axis=1)
    xk = _tc_prepass(x, kernels, bn_gamma.reshape(1, d),
                     bn_beta.reshape(1, d))
    p, hf = _sc_message_pass(xk, ep, hi2, nrel, gpt, rg)
    h = hf.reshape(2, n, nrel)
    escal = jnp.stack([e2[0, 0, 0], e2[0, 0, 2], rl2[0, 0, 0], rl2[0, 0, 2],
                       e2[1, 0, 0], e2[1, 0, 2],
                       jnp.zeros((), jnp.int32), jnp.zeros((), jnp.int32)])
    return _tc_combine(xk, p, h, r, kernels, escal.reshape(8, 1))


# speedup vs baseline: 1.1503x; 1.1503x over previous
import functools

import jax
import jax.numpy as jnp
from jax import lax
from jax.experimental import pallas as pl
from jax.experimental.pallas import tpu as pltpu
from jax.experimental.pallas import tpu_sc as plsc

_C = 128
_SB = 8
_EPS = 1e-5


def _sc_message_pass(xk, ep, hi2, n_cols, gpt, rg):
    n, d = xk.shape
    fl_rows = (n // 16) // 8 * 8
    tail0 = fl_rows * 16
    hflat = n * n_cols

    mesh = plsc.VectorSubcoreMesh(core_axis_name="c", subcore_axis_name="s")

    @functools.partial(
        pl.kernel,
        out_type=(
            jax.ShapeDtypeStruct((2, n, d), jnp.float32),
            jax.ShapeDtypeStruct((2, hflat), jnp.float32),
        ),
        mesh=mesh,
        scratch_types=(
            pltpu.VMEM((2, _SB, _C), jnp.int32),
            pltpu.VMEM((2, _SB, _C), jnp.int32),
            pltpu.VMEM((2, _SB, _C), jnp.int32),
            pltpu.VMEM((2, _C, d), jnp.float32),
            pltpu.VMEM((_C,), jnp.float32),
            pltpu.VMEM((_C,), jnp.float32),
            pltpu.VMEM_SHARED((n, d), jnp.float32),
            pltpu.VMEM_SHARED((hflat,), jnp.float32),
            pltpu.SemaphoreType.DMA((2,)),
            pltpu.SemaphoreType.DMA((2,)),
            pltpu.SemaphoreType.DMA((2,)),
            pltpu.SemaphoreType.DMA,
        ),
    )
    def run(x_hbm, ep_hbm, hi2_hbm, zr_hbm, zh_hbm, pos_hbm, neg_hbm,
            p_hbm, h_hbm,
            src_i, dst_i, hid_i, bufs_v, pos_v, neg_v, acc_sh, hist_sh,
            gsem, ssem, isem, hsem):
        c = lax.axis_index("c")
        s = lax.axis_index("s")
        rbase = s * gpt

        pltpu.sync_copy(pos_hbm, pos_v)
        pltpu.sync_copy(neg_hbm, neg_v)

        z0 = bufs_v.at[0]
        pltpu.sync_copy(zr_hbm, z0)
        zb = s * fl_rows
        nchunks = (fl_rows + _C - 1) // _C
        for k in range(nchunks):
            m = min(_C, fl_rows - k * _C)
            pltpu.sync_copy(z0.at[pl.ds(0, m)],
                            acc_sh.at[pl.ds(zb + k * _C, m)])

        @pl.when(s == 15)
        def _zero_tail():
            left = n - tail0
            off = tail0
            while left > 0:
                mm = min(_C, left)
                pltpu.sync_copy(z0.at[pl.ds(0, mm)],
                                acc_sh.at[pl.ds(off, mm)])
                off += mm
                left -= mm

        @pl.when(s == 0)
        def _zero_hist():
            pltpu.sync_copy(zh_hbm, hist_sh)

        plsc.subcore_barrier()

        pltpu.sync_copy(ep_hbm.at[c, pl.ds(rbase, _SB)], src_i.at[0])
        pltpu.sync_copy(ep_hbm.at[1 - c, pl.ds(rbase, _SB)], dst_i.at[0])
        pltpu.sync_copy(hi2_hbm.at[c, pl.ds(rbase, _SB)], hid_i.at[0])
        pltpu.async_copy(x_hbm.at[src_i.at[0, 0]], bufs_v.at[0], gsem.at[0])

        def body(g, carry):
            sb = g // _SB
            pos = g - sb * _SB
            slot = lax.rem(sb, 2)
            b = lax.rem(g, 2)
            gg = rbase + g

            @pl.when(jnp.logical_and(pos == 0,
                                     jnp.logical_and(g + _SB < gpt,
                                                     gg + _SB < rg)))
            def _prefetch():
                nslot = 1 - slot
                nrow = rbase + (sb + 1) * _SB
                pltpu.async_copy(ep_hbm.at[c, pl.ds(nrow, _SB)],
                                 src_i.at[nslot], isem.at[nslot])
                pltpu.async_copy(ep_hbm.at[1 - c, pl.ds(nrow, _SB)],
                                 dst_i.at[nslot], isem.at[nslot])
                pltpu.async_copy(hi2_hbm.at[c, pl.ds(nrow, _SB)],
                                 hid_i.at[nslot], isem.at[nslot])

            @pl.when(jnp.logical_and(g + 1 < gpt, gg + 1 < rg))
            def _next_gather():
                bn = 1 - b

                @pl.when(g > 0)
                def _wait_prev_scatter():
                    pltpu.make_async_copy(zr_hbm, bufs_v.at[bn],
                                          ssem.at[bn]).wait()

                gn = g + 1
                sbn = gn // _SB
                posn = gn - sbn * _SB
                slotn = lax.rem(sbn, 2)

                @pl.when(posn == 0)
                def _wait_idx():
                    for _ in range(3):
                        pltpu.make_async_copy(ep_hbm.at[c, pl.ds(0, _SB)],
                                              src_i.at[slotn],
                                              isem.at[slotn]).wait()

                pltpu.async_copy(x_hbm.at[src_i.at[slotn, posn]],
                                 bufs_v.at[bn], gsem.at[bn])

            @pl.when(gg < rg)
            def _process():
                pltpu.make_async_copy(x_hbm.at[src_i.at[slot, pos]],
                                      bufs_v.at[b], gsem.at[b]).wait()
                pltpu.async_copy(bufs_v.at[b], acc_sh.at[dst_i.at[slot, pos]],
                                 ssem.at[b], add=True)

                @pl.when(c == 0)
                def _pos_hist():
                    pltpu.async_copy(pos_v, hist_sh.at[hid_i.at[slot, pos]],
                                     hsem, add=True)

                @pl.when(c == 1)
                def _neg_hist():
                    pltpu.async_copy(neg_v, hist_sh.at[hid_i.at[slot, pos]],
                                     hsem, add=True)

            return carry

        lax.fori_loop(0, gpt, body, 0)
        for b in range(2):
            pltpu.make_async_copy(zr_hbm, bufs_v.at[b], ssem.at[b]).wait()
        nreal = jnp.clip(rg - rbase, 0, gpt)

        def _drain(i, carry):
            pltpu.make_async_copy(pos_hbm, pos_v, hsem).wait()
            return carry

        lax.fori_loop(0, nreal, _drain, 0)
        plsc.subcore_barrier()

        fb = s * fl_rows
        pltpu.sync_copy(acc_sh.at[pl.ds(fb, fl_rows)],
                        p_hbm.at[c, pl.ds(fb, fl_rows)])
        pltpu.sync_copy(hist_sh.at[pl.ds(fb * n_cols, fl_rows * n_cols)],
                        h_hbm.at[c, pl.ds(fb * n_cols, fl_rows * n_cols)])

        @pl.when(s == 15)
        def _flush_tail():
            pltpu.sync_copy(acc_sh.at[pl.ds(tail0, n - tail0)],
                            p_hbm.at[c, pl.ds(tail0, n - tail0)])
            pltpu.sync_copy(
                hist_sh.at[pl.ds(tail0 * n_cols, (n - tail0) * n_cols)],
                h_hbm.at[c, pl.ds(tail0 * n_cols, (n - tail0) * n_cols)])

    zr = jnp.zeros((_C, d), jnp.float32)
    zh = jnp.zeros((hflat,), jnp.float32)
    pos1 = jnp.ones((_C,), jnp.float32)
    neg1 = jnp.full((_C,), -1.0, jnp.float32)
    return run(xk, ep, hi2, zr, zh, pos1, neg1)


def _tc_prepass(x, kernels, bn_gamma, bn_beta):
    n, d = x.shape

    def body(x_ref, k_ref, g_ref, b_ref, o_ref):
        xv = x_ref[...]
        mean = jnp.mean(xv, axis=0)
        xc = xv - mean[None, :]
        var = jnp.mean(xc * xc, axis=0)
        sc = g_ref[0, :] / jnp.sqrt(var + _EPS)
        o_ref[...] = (xc * sc[None, :] + b_ref[0, :][None, :]) * k_ref[0, :]

    return pl.pallas_call(
        body,
        out_shape=jax.ShapeDtypeStruct((n, d), jnp.float32),
    )(x, kernels, bn_gamma, bn_beta)


def _tc_combine(xk, p, h, r, kernels, escal):
    n, d = xk.shape

    def body(x_ref, p_ref, h_ref, r_ref, k_ref, es_ref, o_ref):
        rk = r_ref[...] * k_ref[0, :][None, :]
        pv = p_ref[0] + p_ref[1]
        hv = h_ref[0] + h_ref[1]
        relpart = jnp.dot(hv, rk, preferred_element_type=jnp.float32,
                          precision=lax.Precision.HIGHEST)
        num = pv + relpart + x_ref[...]
        esv = es_ref[...]
        iot = lax.broadcasted_iota(jnp.int32, (n, 1), 0)
        du = jnp.ones((n, 1), jnp.float32)
        for i in range(3):
            ai = esv[2 * i:2 * i + 1, :]
            bi = esv[2 * i + 1:2 * i + 2, :]
            inc = (ai != bi).astype(jnp.float32)
            du = du + inc * ((iot == ai).astype(jnp.float32)
                             + (iot == bi).astype(jnp.float32))
        o_ref[...] = num / du

    return pl.pallas_call(
        body,
        out_shape=jax.ShapeDtypeStruct((n, d), jnp.float32),
    )(xk, p, h, r, kernels, escal)


def kernel(x, edges, rels, r, kernels, bn_gamma, bn_beta):
    n, d = x.shape
    e = edges.shape[1]
    nrel = r.shape[0]
    rg = e // _C
    rpad = -(-rg // _SB) * _SB - rg
    gpt = -(-rg // (16 * _SB)) * _SB
    e2 = edges.astype(jnp.int32).reshape(2, rg, _C)
    rl2 = rels.astype(jnp.int32).reshape(1, rg, _C)
    hi2 = jnp.concatenate([e2[1:2], e2[0:1]]) * nrel + rl2
    zpad = jnp.zeros((2, rpad, _C), jnp.int32)
    ep = jnp.concatenate([e2, zpad], axis=1)
    hi2 = jnp.concatenate([hi2, zpad], axis=1)
    xk = _tc_prepass(x, kernels, bn_gamma.reshape(1, d),
                     bn_beta.reshape(1, d))
    p, hf = _sc_message_pass(xk, ep, hi2, nrel, gpt, rg)
    h = hf.reshape(2, n, nrel)
    escal = jnp.stack([e2[0, 0, 0], e2[0, 0, 2], rl2[0, 0, 0], rl2[0, 0, 2],
                       e2[1, 0, 0], e2[1, 0, 2],
                       jnp.zeros((), jnp.int32), jnp.zeros((), jnp.int32)])
    return _tc_combine(xk, p, h, r, kernels, escal.reshape(8, 1))


# speedup vs baseline: 1.2224x; 1.0627x over previous
import functools

import jax
import jax.numpy as jnp
from jax import lax
from jax.experimental import pallas as pl
from jax.experimental.pallas import tpu as pltpu
from jax.experimental.pallas import tpu_sc as plsc

_C = 128
_SB = 8
_EPS = 1e-5


def _sc_message_pass(xk, ep, hi2, n_cols, gpt, rg):
    n, d = xk.shape
    fl_rows = (n // 16) // 8 * 8
    tail0 = fl_rows * 16
    hflat = n * n_cols

    mesh = plsc.VectorSubcoreMesh(core_axis_name="c", subcore_axis_name="s")

    @functools.partial(
        pl.kernel,
        out_type=(
            jax.ShapeDtypeStruct((2, n, d), jnp.float32),
            jax.ShapeDtypeStruct((2, hflat), jnp.float32),
        ),
        mesh=mesh,
        scratch_types=(
            pltpu.VMEM((2, _SB, _C), jnp.int32),
            pltpu.VMEM((2, _SB, _C), jnp.int32),
            pltpu.VMEM((2, _SB, _C), jnp.int32),
            pltpu.VMEM((2, _C, d), jnp.float32),
            pltpu.VMEM((_C,), jnp.float32),
            pltpu.VMEM((_C,), jnp.float32),
            pltpu.VMEM_SHARED((n, d), jnp.float32),
            pltpu.VMEM_SHARED((hflat,), jnp.float32),
            pltpu.SemaphoreType.DMA((2,)),
            pltpu.SemaphoreType.DMA((2,)),
            pltpu.SemaphoreType.DMA((2,)),
            pltpu.SemaphoreType.DMA,
        ),
    )
    def run(x_hbm, ep_hbm, hi2_hbm, zr_hbm, zh_hbm, pos_hbm, neg_hbm,
            p_hbm, h_hbm,
            src_i, dst_i, hid_i, bufs_v, pos_v, neg_v, acc_sh, hist_sh,
            gsem, ssem, isem, hsem):
        c = lax.axis_index("c")
        s = lax.axis_index("s")
        rbase = s * gpt

        pltpu.sync_copy(pos_hbm, pos_v)
        pltpu.sync_copy(neg_hbm, neg_v)

        z0 = bufs_v.at[0]
        pltpu.sync_copy(zr_hbm, z0)
        zb = s * fl_rows
        nchunks = (fl_rows + _C - 1) // _C
        for k in range(nchunks):
            m = min(_C, fl_rows - k * _C)
            pltpu.sync_copy(z0.at[pl.ds(0, m)],
                            acc_sh.at[pl.ds(zb + k * _C, m)])

        @pl.when(s == 15)
        def _zero_tail():
            left = n - tail0
            off = tail0
            while left > 0:
                mm = min(_C, left)
                pltpu.sync_copy(z0.at[pl.ds(0, mm)],
                                acc_sh.at[pl.ds(off, mm)])
                off += mm
                left -= mm

        @pl.when(s == 0)
        def _zero_hist():
            pltpu.sync_copy(zh_hbm, hist_sh)

        plsc.subcore_barrier()

        pltpu.sync_copy(ep_hbm.at[c, pl.ds(rbase, _SB)], src_i.at[0])
        pltpu.sync_copy(ep_hbm.at[1 - c, pl.ds(rbase, _SB)], dst_i.at[0])
        pltpu.sync_copy(hi2_hbm.at[c, pl.ds(rbase, _SB)], hid_i.at[0])
        pltpu.async_copy(x_hbm.at[src_i.at[0, 0]], bufs_v.at[0], gsem.at[0])

        def body(g, carry):
            sb = g // _SB
            pos = g - sb * _SB
            slot = lax.rem(sb, 2)
            b = lax.rem(g, 2)
            gg = rbase + g

            @pl.when(jnp.logical_and(pos == 0,
                                     jnp.logical_and(g + _SB < gpt,
                                                     gg + _SB < rg)))
            def _prefetch():
                nslot = 1 - slot
                nrow = rbase + (sb + 1) * _SB
                pltpu.async_copy(ep_hbm.at[c, pl.ds(nrow, _SB)],
                                 src_i.at[nslot], isem.at[nslot])
                pltpu.async_copy(ep_hbm.at[1 - c, pl.ds(nrow, _SB)],
                                 dst_i.at[nslot], isem.at[nslot])
                pltpu.async_copy(hi2_hbm.at[c, pl.ds(nrow, _SB)],
                                 hid_i.at[nslot], isem.at[nslot])

            @pl.when(jnp.logical_and(g + 1 < gpt, gg + 1 < rg))
            def _next_gather():
                bn = 1 - b

                @pl.when(g > 0)
                def _wait_prev_scatter():
                    pltpu.make_async_copy(zr_hbm, bufs_v.at[bn],
                                          ssem.at[bn]).wait()

                gn = g + 1
                sbn = gn // _SB
                posn = gn - sbn * _SB
                slotn = lax.rem(sbn, 2)

                @pl.when(posn == 0)
                def _wait_idx():
                    for _ in range(3):
                        pltpu.make_async_copy(ep_hbm.at[c, pl.ds(0, _SB)],
                                              src_i.at[slotn],
                                              isem.at[slotn]).wait()

                pltpu.async_copy(x_hbm.at[src_i.at[slotn, posn]],
                                 bufs_v.at[bn], gsem.at[bn])

            @pl.when(gg < rg)
            def _process():
                pltpu.make_async_copy(x_hbm.at[src_i.at[slot, pos]],
                                      bufs_v.at[b], gsem.at[b]).wait()
                pltpu.async_copy(bufs_v.at[b], acc_sh.at[dst_i.at[slot, pos]],
                                 ssem.at[b], add=True)

                @pl.when(c == 0)
                def _pos_hist():
                    pltpu.async_copy(pos_v, hist_sh.at[hid_i.at[slot, pos]],
                                     hsem, add=True)

                @pl.when(c == 1)
                def _neg_hist():
                    pltpu.async_copy(neg_v, hist_sh.at[hid_i.at[slot, pos]],
                                     hsem, add=True)

            return carry

        lax.fori_loop(0, gpt, body, 0)
        for b in range(2):
            pltpu.make_async_copy(zr_hbm, bufs_v.at[b], ssem.at[b]).wait()
        nreal = jnp.clip(rg - rbase, 0, gpt)

        def _drain(i, carry):
            pltpu.make_async_copy(pos_hbm, pos_v, hsem).wait()
            return carry

        lax.fori_loop(0, nreal, _drain, 0)
        plsc.subcore_barrier()

        fb = s * fl_rows
        pltpu.sync_copy(acc_sh.at[pl.ds(fb, fl_rows)],
                        p_hbm.at[c, pl.ds(fb, fl_rows)])
        pltpu.sync_copy(hist_sh.at[pl.ds(fb * n_cols, fl_rows * n_cols)],
                        h_hbm.at[c, pl.ds(fb * n_cols, fl_rows * n_cols)])

        @pl.when(s == 15)
        def _flush_tail():
            pltpu.sync_copy(acc_sh.at[pl.ds(tail0, n - tail0)],
                            p_hbm.at[c, pl.ds(tail0, n - tail0)])
            pltpu.sync_copy(
                hist_sh.at[pl.ds(tail0 * n_cols, (n - tail0) * n_cols)],
                h_hbm.at[c, pl.ds(tail0 * n_cols, (n - tail0) * n_cols)])

    zr = jnp.zeros((_C, d), jnp.float32)
    zh = jnp.zeros((hflat,), jnp.float32)
    pos1 = jnp.ones((_C,), jnp.float32)
    neg1 = jnp.full((_C,), -1.0, jnp.float32)
    return run(xk, ep, hi2, zr, zh, pos1, neg1)


def _tc_prepass(x, e2, rl2, kernels, bn_gamma, bn_beta, nrel, rpad):
    n, d = x.shape
    rg = e2.shape[1]

    def body(x_ref, e_ref, rl_ref, k_ref, g_ref, b_ref,
             o_ref, oep_ref, ohi_ref):
        xv = x_ref[...]
        mean = jnp.mean(xv, axis=0)
        xc = xv - mean[None, :]
        var = jnp.mean(xc * xc, axis=0)
        sc = g_ref[0, :] / jnp.sqrt(var + _EPS)
        o_ref[...] = (xc * sc[None, :] + b_ref[0, :][None, :]) * k_ref[0, :]
        ev = e_ref[...]
        rl = rl_ref[...]
        zp = jnp.zeros((2, rpad, _C), jnp.int32)
        oep_ref[...] = jnp.concatenate([ev, zp], axis=1)
        hi = jnp.concatenate([ev[1:2], ev[0:1]], axis=0) * nrel + rl
        ohi_ref[...] = jnp.concatenate([hi, zp], axis=1)

    return pl.pallas_call(
        body,
        out_shape=(jax.ShapeDtypeStruct((n, d), jnp.float32),
                   jax.ShapeDtypeStruct((2, rg + rpad, _C), jnp.int32),
                   jax.ShapeDtypeStruct((2, rg + rpad, _C), jnp.int32)),
    )(x, e2, rl2, kernels, bn_gamma, bn_beta)


def _tc_combine(xk, p, h, r, kernels, escal):
    n, d = xk.shape

    def body(x_ref, p_ref, h_ref, r_ref, k_ref, es_ref, o_ref):
        rk = r_ref[...] * k_ref[0, :][None, :]
        pv = p_ref[0] + p_ref[1]
        hv = h_ref[0] + h_ref[1]
        relpart = jnp.dot(hv, rk, preferred_element_type=jnp.float32,
                          precision=lax.Precision.HIGHEST)
        num = pv + relpart + x_ref[...]
        esv = es_ref[...]
        iot = lax.broadcasted_iota(jnp.int32, (n, 1), 0)
        du = jnp.ones((n, 1), jnp.float32)
        for i in range(3):
            ai = esv[2 * i:2 * i + 1, :]
            bi = esv[2 * i + 1:2 * i + 2, :]
            inc = (ai != bi).astype(jnp.float32)
            du = du + inc * ((iot == ai).astype(jnp.float32)
                             + (iot == bi).astype(jnp.float32))
        o_ref[...] = num / du

    return pl.pallas_call(
        body,
        out_shape=jax.ShapeDtypeStruct((n, d), jnp.float32),
    )(xk, p, h, r, kernels, escal)


def kernel(x, edges, rels, r, kernels, bn_gamma, bn_beta):
    n, d = x.shape
    e = edges.shape[1]
    nrel = r.shape[0]
    rg = e // _C
    rpad = -(-rg // _SB) * _SB - rg
    gpt = -(-rg // (16 * _SB)) * _SB
    e2 = edges.astype(jnp.int32).reshape(2, rg, _C)
    rl2 = rels.astype(jnp.int32).reshape(1, rg, _C)
    xk, ep, hi2 = _tc_prepass(x, e2, rl2, kernels, bn_gamma.reshape(1, d),
                              bn_beta.reshape(1, d), nrel, rpad)
    p, hf = _sc_message_pass(xk, ep, hi2, nrel, gpt, rg)
    h = hf.reshape(2, n, nrel)
    escal = jnp.stack([e2[0, 0, 0], e2[0, 0, 2], rl2[0, 0, 0], rl2[0, 0, 2],
                       e2[1, 0, 0], e2[1, 0, 2],
                       jnp.zeros((), jnp.int32), jnp.zeros((), jnp.int32)])
    return _tc_combine(xk, p, h, r, kernels, escal.reshape(8, 1))


# speedup vs baseline: 1.2428x; 1.0166x over previous
import functools

import jax
import jax.numpy as jnp
from jax import lax
from jax.experimental import pallas as pl
from jax.experimental.pallas import tpu as pltpu
from jax.experimental.pallas import tpu_sc as plsc

_C = 128
_SB = 8
_EPS = 1e-5


def _sc_message_pass(xk, ep, hi2, n_cols, gpt, rg):
    n, d = xk.shape
    fl_rows = (n // 16) // 8 * 8
    tail0 = fl_rows * 16
    hflat = n * n_cols

    mesh = plsc.VectorSubcoreMesh(core_axis_name="c", subcore_axis_name="s")

    @functools.partial(
        pl.kernel,
        out_type=(
            jax.ShapeDtypeStruct((2, n, d), jnp.float32),
            jax.ShapeDtypeStruct((2, hflat), jnp.float32),
        ),
        mesh=mesh,
        scratch_types=(
            pltpu.VMEM((2, _SB, _C), jnp.int32),
            pltpu.VMEM((2, _SB, _C), jnp.int32),
            pltpu.VMEM((2, _SB, _C), jnp.int32),
            pltpu.VMEM((2, _C, d), jnp.float32),
            pltpu.VMEM((_C,), jnp.float32),
            pltpu.VMEM((_C,), jnp.float32),
            pltpu.VMEM_SHARED((n, d), jnp.float32),
            pltpu.VMEM_SHARED((hflat,), jnp.float32),
            pltpu.SemaphoreType.DMA((2,)),
            pltpu.SemaphoreType.DMA((2,)),
            pltpu.SemaphoreType.DMA((2,)),
            pltpu.SemaphoreType.DMA,
        ),
    )
    def run(x_hbm, ep_hbm, hi2_hbm, zr_hbm, zh_hbm, pos_hbm, neg_hbm,
            p_hbm, h_hbm,
            src_i, dst_i, hid_i, bufs_v, pos_v, neg_v, acc_sh, hist_sh,
            gsem, ssem, isem, hsem):
        c = lax.axis_index("c")
        s = lax.axis_index("s")
        rbase = s * gpt

        pltpu.sync_copy(pos_hbm, pos_v)
        pltpu.sync_copy(neg_hbm, neg_v)

        z0 = bufs_v.at[0]
        pltpu.sync_copy(zr_hbm, z0)
        zb = s * fl_rows
        nchunks = (fl_rows + _C - 1) // _C
        for k in range(nchunks):
            m = min(_C, fl_rows - k * _C)
            pltpu.sync_copy(z0.at[pl.ds(0, m)],
                            acc_sh.at[pl.ds(zb + k * _C, m)])

        @pl.when(s == 15)
        def _zero_tail():
            left = n - tail0
            off = tail0
            while left > 0:
                mm = min(_C, left)
                pltpu.sync_copy(z0.at[pl.ds(0, mm)],
                                acc_sh.at[pl.ds(off, mm)])
                off += mm
                left -= mm

        @pl.when(s == 0)
        def _zero_hist():
            pltpu.sync_copy(zh_hbm, hist_sh)

        plsc.subcore_barrier()

        pltpu.sync_copy(ep_hbm.at[c, pl.ds(rbase, _SB)], src_i.at[0])
        pltpu.sync_copy(ep_hbm.at[1 - c, pl.ds(rbase, _SB)], dst_i.at[0])
        pltpu.sync_copy(hi2_hbm.at[c, pl.ds(rbase, _SB)], hid_i.at[0])
        pltpu.async_copy(x_hbm.at[src_i.at[0, 0]], bufs_v.at[0], gsem.at[0])

        def body(g, carry):
            sb = g // _SB
            pos = g - sb * _SB
            slot = lax.rem(sb, 2)
            b = lax.rem(g, 2)
            gg = rbase + g

            @pl.when(jnp.logical_and(pos == 0,
                                     jnp.logical_and(g + _SB < gpt,
                                                     gg + _SB < rg)))
            def _prefetch():
                nslot = 1 - slot
                nrow = rbase + (sb + 1) * _SB
                pltpu.async_copy(ep_hbm.at[c, pl.ds(nrow, _SB)],
                                 src_i.at[nslot], isem.at[nslot])
                pltpu.async_copy(ep_hbm.at[1 - c, pl.ds(nrow, _SB)],
                                 dst_i.at[nslot], isem.at[nslot])
                pltpu.async_copy(hi2_hbm.at[c, pl.ds(nrow, _SB)],
                                 hid_i.at[nslot], isem.at[nslot])

            @pl.when(jnp.logical_and(g + 1 < gpt, gg + 1 < rg))
            def _next_gather():
                bn = 1 - b

                @pl.when(g > 0)
                def _wait_prev_scatter():
                    pltpu.make_async_copy(zr_hbm, bufs_v.at[bn],
                                          ssem.at[bn]).wait()

                gn = g + 1
                sbn = gn // _SB
                posn = gn - sbn * _SB
                slotn = lax.rem(sbn, 2)

                @pl.when(posn == 0)
                def _wait_idx():
                    for _ in range(3):
                        pltpu.make_async_copy(ep_hbm.at[c, pl.ds(0, _SB)],
                                              src_i.at[slotn],
                                              isem.at[slotn]).wait()

                pltpu.async_copy(x_hbm.at[src_i.at[slotn, posn]],
                                 bufs_v.at[bn], gsem.at[bn])

            @pl.when(gg < rg)
            def _process():
                pltpu.make_async_copy(x_hbm.at[src_i.at[slot, pos]],
                                      bufs_v.at[b], gsem.at[b]).wait()
                pltpu.async_copy(bufs_v.at[b], acc_sh.at[dst_i.at[slot, pos]],
                                 ssem.at[b], add=True)

                @pl.when(c == 0)
                def _pos_hist():
                    pltpu.async_copy(pos_v, hist_sh.at[hid_i.at[slot, pos]],
                                     hsem, add=True)

                @pl.when(c == 1)
                def _neg_hist():
                    pltpu.async_copy(neg_v, hist_sh.at[hid_i.at[slot, pos]],
                                     hsem, add=True)

            return carry

        lax.fori_loop(0, gpt, body, 0)
        for b in range(2):
            pltpu.make_async_copy(zr_hbm, bufs_v.at[b], ssem.at[b]).wait()
        nreal = jnp.clip(rg - rbase, 0, gpt)

        def _drain(i, carry):
            pltpu.make_async_copy(pos_hbm, pos_v, hsem).wait()
            return carry

        lax.fori_loop(0, nreal, _drain, 0)
        plsc.subcore_barrier()

        fb = s * fl_rows
        pltpu.sync_copy(acc_sh.at[pl.ds(fb, fl_rows)],
                        p_hbm.at[c, pl.ds(fb, fl_rows)])
        pltpu.sync_copy(hist_sh.at[pl.ds(fb * n_cols, fl_rows * n_cols)],
                        h_hbm.at[c, pl.ds(fb * n_cols, fl_rows * n_cols)])

        @pl.when(s == 15)
        def _flush_tail():
            pltpu.sync_copy(acc_sh.at[pl.ds(tail0, n - tail0)],
                            p_hbm.at[c, pl.ds(tail0, n - tail0)])
            pltpu.sync_copy(
                hist_sh.at[pl.ds(tail0 * n_cols, (n - tail0) * n_cols)],
                h_hbm.at[c, pl.ds(tail0 * n_cols, (n - tail0) * n_cols)])

    zr = jnp.zeros((_C, d), jnp.float32)
    zh = jnp.zeros((hflat,), jnp.float32)
    pos1 = jnp.ones((_C,), jnp.float32)
    neg1 = jnp.full((_C,), -1.0, jnp.float32)
    return run(xk, ep, hi2, zr, zh, pos1, neg1)


def _tc_prepass(x, e2, rl2, kernels, bn_gamma, bn_beta, nrel, rpad):
    n, d = x.shape
    rg = e2.shape[1]

    def body(x_ref, e_ref, rl_ref, k_ref, g_ref, b_ref,
             o_ref, oep_ref, ohi_ref):
        xv = x_ref[...]
        mean = jnp.mean(xv, axis=0)
        xc = xv - mean[None, :]
        var = jnp.mean(xc * xc, axis=0)
        sc = g_ref[0, :] / jnp.sqrt(var + _EPS)
        o_ref[...] = (xc * sc[None, :] + b_ref[0, :][None, :]) * k_ref[0, :]
        ev = e_ref[...]
        rl = rl_ref[...]
        zp = jnp.zeros((2, rpad, _C), jnp.int32)
        oep_ref[...] = jnp.concatenate([ev, zp], axis=1)
        hi = jnp.concatenate([ev[1:2], ev[0:1]], axis=0) * nrel + rl
        ohi_ref[...] = jnp.concatenate([hi, zp], axis=1)

    return pl.pallas_call(
        body,
        out_shape=(jax.ShapeDtypeStruct((n, d), jnp.float32),
                   jax.ShapeDtypeStruct((2, rg + rpad, _C), jnp.int32),
                   jax.ShapeDtypeStruct((2, rg + rpad, _C), jnp.int32)),
    )(x, e2, rl2, kernels, bn_gamma, bn_beta)


def _tc_combine(xk, p, h, r, kernels, escal):
    n, d = xk.shape
    nc = h.shape[2]
    bs = 2000
    grid = n // bs

    def body(x_ref, p_ref, h_ref, r_ref, k_ref, es_ref, o_ref):
        rk = r_ref[...] * k_ref[0, :][None, :]
        pv = p_ref[0] + p_ref[1]
        hv = h_ref[0] + h_ref[1]
        relpart = jnp.dot(hv, rk, preferred_element_type=jnp.float32,
                          precision=lax.Precision.HIGHEST)
        num = pv + relpart + x_ref[...]
        esv = es_ref[...]
        iot = (lax.broadcasted_iota(jnp.int32, (bs, 1), 0)
               + pl.program_id(0) * bs)
        du = jnp.ones((bs, 1), jnp.float32)
        for i in range(3):
            ai = esv[2 * i:2 * i + 1, :]
            bi = esv[2 * i + 1:2 * i + 2, :]
            inc = (ai != bi).astype(jnp.float32)
            du = du + inc * ((iot == ai).astype(jnp.float32)
                             + (iot == bi).astype(jnp.float32))
        o_ref[...] = num / du

    return pl.pallas_call(
        body,
        grid=(grid,),
        in_specs=[
            pl.BlockSpec((bs, d), lambda i: (i, 0)),
            pl.BlockSpec((2, bs, d), lambda i: (0, i, 0)),
            pl.BlockSpec((2, bs, nc), lambda i: (0, i, 0)),
            pl.BlockSpec(r.shape, lambda i: (0, 0)),
            pl.BlockSpec(kernels.shape, lambda i: (0, 0)),
            pl.BlockSpec(escal.shape, lambda i: (0, 0)),
        ],
        out_specs=pl.BlockSpec((bs, d), lambda i: (i, 0)),
        out_shape=jax.ShapeDtypeStruct((n, d), jnp.float32),
    )(xk, p, h, r, kernels, escal)


def kernel(x, edges, rels, r, kernels, bn_gamma, bn_beta):
    n, d = x.shape
    e = edges.shape[1]
    nrel = r.shape[0]
    rg = e // _C
    rpad = -(-rg // _SB) * _SB - rg
    gpt = -(-rg // (16 * _SB)) * _SB
    e2 = edges.astype(jnp.int32).reshape(2, rg, _C)
    rl2 = rels.astype(jnp.int32).reshape(1, rg, _C)
    xk, ep, hi2 = _tc_prepass(x, e2, rl2, kernels, bn_gamma.reshape(1, d),
                              bn_beta.reshape(1, d), nrel, rpad)
    p, hf = _sc_message_pass(xk, ep, hi2, nrel, gpt, rg)
    h = hf.reshape(2, n, nrel)
    escal = jnp.stack([e2[0, 0, 0], e2[0, 0, 2], rl2[0, 0, 0], rl2[0, 0, 2],
                       e2[1, 0, 0], e2[1, 0, 2],
                       jnp.zeros((), jnp.int32), jnp.zeros((), jnp.int32)])
    return _tc_combine(xk, p, h, r, kernels, escal.reshape(8, 1))
